# baseline (speedup 1.0000x reference)
import jax
import jax.numpy as jnp
from jax.experimental import pallas as pl
from jax.experimental.pallas import tpu as pltpu

E_ = 8
TD = 512
TC = 256
ND = 2048 // TD
NC = 2048 // TC


def _moe_body(x_ref, disp_ref, comb_ref, w_ref, b_ref, out_ref, xd_acc, y_s, w_s):
    t = pl.program_id(0)

    @pl.when(t < ND)
    def _dispatch_phase():
        xt = jnp.transpose(x_ref[0].astype(jnp.bfloat16), (1, 0, 2))
        dt = jnp.transpose(disp_ref[0].astype(jnp.bfloat16), (1, 0, 2))

        chunk = w_ref[...].astype(jnp.bfloat16).reshape(2 * 768, 256)
        w_s[2 * t] = chunk[:768]
        w_s[2 * t + 1] = chunk[768:]

        for e in range(E_):
            part = jax.lax.dot_general(
                dt[e], xt[e], (((0,), (0,)), ((), ())),
                preferred_element_type=jnp.float32,
            )

            @pl.when(t == 0)
            def _set():
                xd_acc[e] = part

            @pl.when(t > 0)
            def _add():
                xd_acc[e] += part

        @pl.when(t == ND - 1)
        def _expert_matmul():
            for e in range(E_):
                y = jax.lax.dot_general(
                    xd_acc[e].astype(jnp.bfloat16), w_s[e],
                    (((1,), (1,)), ((), ())),
                    preferred_element_type=jnp.float32,
                )
                y_s[e] = (y + b_ref[...]).astype(jnp.bfloat16)

    @pl.when(t >= ND)
    def _combine_phase():
        ct = jnp.transpose(comb_ref[0].astype(jnp.bfloat16), (1, 0, 2))
        acc = jnp.zeros((TC, 768), jnp.float32)
        for e in range(E_):
            acc += jnp.dot(ct[e], y_s[e], preferred_element_type=jnp.float32)
        out_ref[...] = acc


def kernel(x, combine_array, dispatch_mask, W, b):
    B, T, E, I = x.shape
    C = combine_array.shape[-1]
    O = W.shape[0]
    b2 = b.reshape(1, O)

    out = pl.pallas_call(
        _moe_body,
        grid=(ND + NC,),
        in_specs=[
            pl.BlockSpec((1, TD, E, I),
                         lambda t: (0, jnp.where(t < ND, t, ND - 1), 0, 0)),
            pl.BlockSpec((1, TD, E, C),
                         lambda t: (0, jnp.where(t < ND, t, ND - 1), 0, 0)),
            pl.BlockSpec((1, TC, E, C),
                         lambda t: (0, jnp.where(t < ND, 0, t - ND), 0, 0)),
            pl.BlockSpec((192, E * I),
                         lambda t: (jnp.where(t < ND, t, ND - 1), 0)),
            pl.BlockSpec((1, O), lambda t: (0, 0)),
        ],
        out_specs=pl.BlockSpec((TC, O), lambda t: (jnp.where(t < ND, 0, t - ND), 0)),
        out_shape=jax.ShapeDtypeStruct((T, O), jnp.float32),
        scratch_shapes=[
            pltpu.VMEM((E_, 256, 256), jnp.float32),
            pltpu.VMEM((E_, 256, 768), jnp.bfloat16),
            pltpu.VMEM((E_, 768, 256), jnp.bfloat16),
        ],
        compiler_params=pltpu.CompilerParams(
            dimension_semantics=("arbitrary",),
        ),
    )(x, dispatch_mask, combine_array, W, b2)
    return out.reshape(B, T, O)


# speedup vs baseline: 1.1973x; 1.1973x over previous
import jax
import jax.numpy as jnp
from jax.experimental import pallas as pl
from jax.experimental.pallas import tpu as pltpu

E_ = 8
TILE_T = 512


def _moe_body(x_ref, disp_ref, comb_ref, w_ref, b_ref, out_ref, xd_acc, y_s, w_s):
    p = pl.program_id(0)
    t = pl.program_id(1)
    nt = pl.num_programs(1)

    @pl.when(p == 0)
    def _dispatch_phase():
        xt = jnp.transpose(x_ref[0].astype(jnp.bfloat16), (1, 0, 2))
        dt = jnp.transpose(disp_ref[0].astype(jnp.bfloat16), (1, 0, 2))

        @pl.when(t == 0)
        def _init():
            xd_acc[...] = jnp.zeros_like(xd_acc)

        chunk = w_ref[...].astype(jnp.bfloat16).reshape(2 * 768, 256)
        w_s[2 * t] = chunk[:768]
        w_s[2 * t + 1] = chunk[768:]

        for e in range(E_):
            xd_acc[e] += jax.lax.dot_general(
                dt[e], xt[e], (((0,), (0,)), ((), ())),
                preferred_element_type=jnp.float32,
            )

        @pl.when(t == nt - 1)
        def _expert_matmul():
            for e in range(E_):
                y = jax.lax.dot_general(
                    xd_acc[e].astype(jnp.bfloat16), w_s[e],
                    (((1,), (1,)), ((), ())),
                    preferred_element_type=jnp.float32,
                )
                y_s[e] = (y + b_ref[...]).astype(jnp.bfloat16)

    @pl.when(p == 1)
    def _combine_phase():
        ct = jnp.transpose(comb_ref[0].astype(jnp.bfloat16), (1, 0, 2))
        acc = jnp.zeros(out_ref.shape, jnp.float32)
        for e in range(E_):
            acc += jnp.dot(ct[e], y_s[e], preferred_element_type=jnp.float32)
        out_ref[...] = acc


def kernel(x, combine_array, dispatch_mask, W, b):
    B, T, E, I = x.shape
    C = combine_array.shape[-1]
    O = W.shape[0]
    nt = T // TILE_T
    b2 = b.reshape(1, O)

    out = pl.pallas_call(
        _moe_body,
        grid=(2, nt),
        in_specs=[
            pl.BlockSpec((1, TILE_T, E, I),
                         lambda p, t: (0, jnp.where(p == 0, t, nt - 1), 0, 0)),
            pl.BlockSpec((1, TILE_T, E, C),
                         lambda p, t: (0, jnp.where(p == 0, t, nt - 1), 0, 0)),
            pl.BlockSpec((1, TILE_T, E, C),
                         lambda p, t: (0, jnp.where(p == 0, 0, t), 0, 0)),
            pl.BlockSpec((192, E * I),
                         lambda p, t: (jnp.where(p == 0, t, nt - 1), 0)),
            pl.BlockSpec((1, O), lambda p, t: (0, 0)),
        ],
        out_specs=pl.BlockSpec((TILE_T, O), lambda p, t: (jnp.where(p == 0, 0, t), 0)),
        out_shape=jax.ShapeDtypeStruct((T, O), jnp.float32),
        scratch_shapes=[
            pltpu.VMEM((E_, 256, 256), jnp.float32),
            pltpu.VMEM((E_, 256, 768), jnp.bfloat16),
            pltpu.VMEM((E_, 768, 256), jnp.bfloat16),
        ],
        compiler_params=pltpu.CompilerParams(
            dimension_semantics=("arbitrary", "arbitrary"),
        ),
    )(x, dispatch_mask, combine_array, W, b2)
    return out.reshape(B, T, O)
